# baseline (speedup 1.0000x reference)
import jax
import jax.numpy as jnp
from jax.experimental import pallas as pl

_BB = 128


def _bcast_body(pe_ref, out_ref):
    out_ref[...] = jnp.broadcast_to(pe_ref[...][None, :, :], out_ref.shape)


def kernel(x, pe_weight):
    batch = x.shape[0]
    max_len, d_model = pe_weight.shape
    bb = _BB if batch % _BB == 0 else 1
    return pl.pallas_call(
        _bcast_body,
        grid=(batch // bb,),
        in_specs=[pl.BlockSpec((max_len, d_model), lambda i: (0, 0))],
        out_specs=pl.BlockSpec((bb, max_len, d_model), lambda i: (i, 0, 0)),
        out_shape=jax.ShapeDtypeStruct((batch, max_len, d_model), pe_weight.dtype),
    )(pe_weight)


# speedup vs baseline: 1.0473x; 1.0473x over previous
import jax
import jax.numpy as jnp
from jax.experimental import pallas as pl

_BB = 32


def _bcast_body(pe_ref, out_ref):
    out_ref[...] = jnp.broadcast_to(pe_ref[...][None, :, :], out_ref.shape)


def kernel(x, pe_weight):
    batch = x.shape[0]
    max_len, d_model = pe_weight.shape
    bb = _BB if batch % _BB == 0 else 1
    return pl.pallas_call(
        _bcast_body,
        grid=(batch // bb,),
        in_specs=[pl.BlockSpec((max_len, d_model), lambda i: (0, 0))],
        out_specs=pl.BlockSpec((bb, max_len, d_model), lambda i: (i, 0, 0)),
        out_shape=jax.ShapeDtypeStruct((batch, max_len, d_model), pe_weight.dtype),
    )(pe_weight)
